# Initial kernel scaffold; baseline (speedup 1.0000x reference)
import jax
import jax.numpy as jnp
from jax import lax
from jax.experimental import pallas as pl
from jax.experimental.pallas import tpu as pltpu
from jax.experimental.pallas import tpu_sc as plsc

N_PAPER = 50000
N_AUTHOR = 10000
D = 128
E = 320000

NC = 2
NS = 16
LANES = 16

CHUNK_C = 12544
N_CHUNKS_C = 4
PAD_PAPER = CHUNK_C * N_CHUNKS_C
WPT_C = CHUNK_C // NS
CHUNK_B = 10112
WPT_B = CHUNK_B // NS
ACC_ROWS = 12800
ZPT = ACC_ROWS // NS
TRASH_C = CHUNK_C
TRASH_B = CHUNK_B

BLK = 2048
BATCH = 128
PER_TILE = E // NS


def _zero_vmem_2d(ref, rows):
    zv = jnp.zeros((LANES,), jnp.float32)

    def body(i, _):
        for h in range(D // LANES):
            ref[i, pl.ds(h * LANES, LANES)] = zv
        return 0

    lax.fori_loop(0, rows, body, 0)


def _zero_vmem_1d(ref, n):
    zv = jnp.zeros((LANES,), jnp.float32)

    def body(i, _):
        ref[pl.ds(i * LANES, LANES)] = zv
        return 0

    lax.fori_loop(0, n // LANES, body, 0)


def _fill_ones_1d(ref, n):
    ov = jnp.ones((LANES,), jnp.float32)
    for i in range(n // LANES):
        ref[pl.ds(i * LANES, LANES)] = ov


def _zero_pass(sid, n_acc_rows, rows_v, zvec, acc_sh, cnt_sh):
    _zero_vmem_2d(rows_v, BATCH)
    _zero_vmem_1d(zvec, ZPT)
    zpt = n_acc_rows // NS
    r0 = sid * zpt
    done = 0
    while done < zpt:
        n = min(BATCH, zpt - done)
        pltpu.sync_copy(rows_v.at[pl.ds(0, n), :], acc_sh.at[pl.ds(r0 + done, n), :])
        done += n
    pltpu.sync_copy(zvec.at[pl.ds(0, zpt)], cnt_sh.at[pl.ds(r0, zpt)])


def _accumulate(sid, x_hbm, e_hbm, lo, hi, trash,
                src_blk, dst_blk, idx_src, idx_feat, ones_v, rows_v,
                acc_sh, cnt_sh, sem):
    e_tile = sid * PER_TILE

    def do_batch(off, ngroups):
        for g in range(ngroups):
            s_v = src_blk[pl.ds(off + g * LANES, LANES)]
            d_v = dst_blk[pl.ds(off + g * LANES, LANES)]
            inb = (d_v >= lo) & (d_v < hi)
            eff = jnp.where(inb, d_v - lo, trash)
            idx_src[pl.ds(g * LANES, LANES)] = s_v
            idx_feat[pl.ds(g * LANES, LANES)] = eff
        for g in range(ngroups, BATCH // LANES):
            idx_src[pl.ds(g * LANES, LANES)] = jnp.zeros((LANES,), jnp.int32)
            idx_feat[pl.ds(g * LANES, LANES)] = jnp.full((LANES,), trash, jnp.int32)
        pltpu.async_copy(x_hbm.at[idx_src], rows_v, sem).wait()
        pltpu.sync_copy(rows_v, acc_sh.at[idx_feat], add=True)
        pltpu.sync_copy(ones_v, cnt_sh.at[idx_feat], add=True)

    def do_block(e0, bs):
        pltpu.sync_copy(e_hbm.at[0, pl.ds(e0, bs)], src_blk.at[pl.ds(0, bs)])
        pltpu.sync_copy(e_hbm.at[1, pl.ds(e0, bs)], dst_blk.at[pl.ds(0, bs)])
        nb = bs // BATCH

        def bbody(b, _):
            do_batch(b * BATCH, BATCH // LANES)
            return 0

        lax.fori_loop(0, nb, bbody, 0)
        tail = bs % BATCH
        if tail:
            do_batch(nb * BATCH, tail // LANES)

    n_full = PER_TILE // BLK

    def blk_body(k, _):
        do_block(e_tile + k * BLK, BLK)
        return 0

    lax.fori_loop(0, n_full, blk_body, 0)
    rem = PER_TILE % BLK
    if rem:
        do_block(e_tile + n_full * BLK, rem)


def _divide_writeout(sid, wpt, out_hbm, out_base,
                     rows_v, cnt_stage, inv_stage, acc_sh, cnt_sh):
    r0 = sid * wpt
    pltpu.sync_copy(cnt_sh.at[pl.ds(r0, wpt)], cnt_stage.at[pl.ds(0, wpt)])

    def inv_body(g, _):
        c_v = cnt_stage[pl.ds(g * LANES, LANES)]
        inv_stage[pl.ds(g * LANES, LANES)] = 1.0 / jnp.maximum(c_v, 1.0)
        return 0

    lax.fori_loop(0, wpt // LANES, inv_body, 0)

    lane_iota = lax.iota(jnp.int32, LANES)
    done = 0
    while done < wpt:
        n = min(BATCH, wpt - done)
        pltpu.sync_copy(acc_sh.at[pl.ds(r0 + done, n), :], rows_v.at[pl.ds(0, n), :])
        base = done

        def grp_body(g, _):
            inv_v = inv_stage[pl.ds(base + g * LANES, LANES)]
            for j in range(LANES):
                s = jnp.sum(jnp.where(lane_iota == j, inv_v, 0.0))
                row = g * LANES + j
                for h in range(D // LANES):
                    rows_v[row, pl.ds(h * LANES, LANES)] = (
                        rows_v[row, pl.ds(h * LANES, LANES)] * s)
            return 0

        lax.fori_loop(0, n // LANES, grp_body, 0)
        pltpu.sync_copy(rows_v.at[pl.ds(0, n), :],
                        out_hbm.at[pl.ds(out_base + r0 + done, n), :])
        done += n


def _sc_body(x_paper, x_author, e_c, e_w, e_wb,
             mean_c, mean_w, mean_wb,
             src_blk, dst_blk, idx_src, idx_feat, ones_v, zvec,
             rows_v, cnt_stage, inv_stage, acc_sh, cnt_sh, sem):
    cid = lax.axis_index("c")
    sid = lax.axis_index("s")
    _fill_ones_1d(ones_v, BATCH)

    for p in range(2):
        lo = (2 * cid + p) * CHUNK_C
        hi = lo + CHUNK_C
        _zero_pass(sid, ACC_ROWS, rows_v, zvec, acc_sh, cnt_sh)
        plsc.subcore_barrier()
        _accumulate(sid, x_paper, e_c, lo, hi, TRASH_C,
                    src_blk, dst_blk, idx_src, idx_feat, ones_v, rows_v,
                    acc_sh, cnt_sh, sem)
        plsc.subcore_barrier()
        _divide_writeout(sid, WPT_C, mean_c, lo,
                         rows_v, cnt_stage, inv_stage, acc_sh, cnt_sh)
        plsc.subcore_barrier()

    _zero_pass(sid, ACC_ROWS, rows_v, zvec, acc_sh, cnt_sh)
    plsc.subcore_barrier()

    @pl.when(cid == 0)
    def _():
        _accumulate(sid, x_author, e_w, 0, CHUNK_B, TRASH_B,
                    src_blk, dst_blk, idx_src, idx_feat, ones_v, rows_v,
                    acc_sh, cnt_sh, sem)

    @pl.when(cid == 1)
    def _():
        _accumulate(sid, x_paper, e_wb, 0, CHUNK_B, TRASH_B,
                    src_blk, dst_blk, idx_src, idx_feat, ones_v, rows_v,
                    acc_sh, cnt_sh, sem)

    plsc.subcore_barrier()

    @pl.when(cid == 0)
    def _():
        _divide_writeout(sid, WPT_B, mean_w, 0,
                         rows_v, cnt_stage, inv_stage, acc_sh, cnt_sh)

    @pl.when(cid == 1)
    def _():
        _divide_writeout(sid, WPT_B, mean_wb, 0,
                         rows_v, cnt_stage, inv_stage, acc_sh, cnt_sh)


def _segment_means(x_paper, x_author, e_c, e_w, e_wb):
    mesh = plsc.VectorSubcoreMesh(core_axis_name="c", subcore_axis_name="s")
    f32 = jnp.float32
    kfn = pl.kernel(
        _sc_body,
        out_type=(
            jax.ShapeDtypeStruct((PAD_PAPER, D), f32),
            jax.ShapeDtypeStruct((CHUNK_B, D), f32),
            jax.ShapeDtypeStruct((CHUNK_B, D), f32),
        ),
        mesh=mesh,
        scratch_types=[
            pltpu.VMEM((BLK,), jnp.int32),
            pltpu.VMEM((BLK,), jnp.int32),
            pltpu.VMEM((BATCH,), jnp.int32),
            pltpu.VMEM((BATCH,), jnp.int32),
            pltpu.VMEM((BATCH,), f32),
            pltpu.VMEM((ZPT,), f32),
            pltpu.VMEM((BATCH, D), f32),
            pltpu.VMEM((ZPT,), f32),
            pltpu.VMEM((ZPT,), f32),
            pltpu.VMEM_SHARED((ACC_ROWS, D), f32),
            pltpu.VMEM_SHARED((ACC_ROWS,), f32),
            pltpu.SemaphoreType.DMA,
        ],
    )
    return kfn(x_paper, x_author, e_c, e_w, e_wb)


BR = 400
N_W_BLOCKS = N_AUTHOR // BR


def _tc_paper_body(x_ref, mc_ref, mw_ref, wr_ref, wnc_ref, wnw_ref, b_ref, o_ref):
    i = pl.program_id(0)
    out = jnp.dot(x_ref[...], wr_ref[...], preferred_element_type=jnp.float32)
    out = out + jnp.dot(mc_ref[...], wnc_ref[...], preferred_element_type=jnp.float32)
    o_ref[...] = out + b_ref[...]

    @pl.when(i < N_W_BLOCKS)
    def _():
        o_ref[...] = o_ref[...] + jnp.dot(
            mw_ref[...], wnw_ref[...], preferred_element_type=jnp.float32)


def _tc_author_body(x_ref, m_ref, wr_ref, wn_ref, b_ref, o_ref):
    out = jnp.dot(x_ref[...], wr_ref[...], preferred_element_type=jnp.float32)
    out = out + jnp.dot(m_ref[...], wn_ref[...], preferred_element_type=jnp.float32)
    o_ref[...] = out + b_ref[...]


def _wspec():
    return pl.BlockSpec((D, D), lambda i: (0, 0))


def _bspec():
    return pl.BlockSpec((1, D), lambda i: (0, 0))


def _tc_paper(x_paper, mean_c, mean_w, wr, wnc, wnw, b):
    grid = (N_PAPER // BR,)
    return pl.pallas_call(
        _tc_paper_body,
        grid=grid,
        in_specs=[
            pl.BlockSpec((BR, D), lambda i: (i, 0)),
            pl.BlockSpec((BR, D), lambda i: (i, 0)),
            pl.BlockSpec((BR, D), lambda i: (jnp.minimum(i, N_W_BLOCKS - 1), 0)),
            _wspec(), _wspec(), _wspec(), _bspec(),
        ],
        out_specs=pl.BlockSpec((BR, D), lambda i: (i, 0)),
        out_shape=jax.ShapeDtypeStruct((N_PAPER, D), jnp.float32),
    )(x_paper, mean_c, mean_w, wr, wnc, wnw, b)


def _tc_author(x_author, mean_wb, wr, wn, b):
    grid = (N_AUTHOR // BR,)
    return pl.pallas_call(
        _tc_author_body,
        grid=grid,
        in_specs=[
            pl.BlockSpec((BR, D), lambda i: (i, 0)),
            pl.BlockSpec((BR, D), lambda i: (i, 0)),
            _wspec(), _wspec(), _bspec(),
        ],
        out_specs=pl.BlockSpec((BR, D), lambda i: (i, 0)),
        out_shape=jax.ShapeDtypeStruct((N_AUTHOR, D), jnp.float32),
    )(x_author, mean_wb, wr, wn, b)


@jax.jit
def kernel(x_paper, x_author, edge_index_cites, edge_index_writes,
           edge_index_written_by,
           W_root_cites, W_neigh_cites, b_cites,
           W_root_writes, W_neigh_writes, b_writes,
           W_root_wb, W_neigh_wb, b_wb):
    mean_c, mean_w, mean_wb = _segment_means(
        x_paper, x_author, edge_index_cites, edge_index_writes,
        edge_index_written_by)
    out_paper = _tc_paper(
        x_paper, mean_c, mean_w,
        W_root_cites + W_root_writes, W_neigh_cites, W_neigh_writes,
        (b_cites + b_writes).reshape(1, D))
    out_author = _tc_author(
        x_author, mean_wb, W_root_wb, W_neigh_wb, b_wb.reshape(1, D))
    return (out_paper, out_author)


# speedup vs baseline: 1.7435x; 1.7435x over previous
import jax
import jax.numpy as jnp
from jax import lax
from jax.experimental import pallas as pl
from jax.experimental.pallas import tpu as pltpu
from jax.experimental.pallas import tpu_sc as plsc

N_PAPER = 50000
N_AUTHOR = 10000
D = 128
E = 320000

NC = 2
NS = 16
LANES = 16

CHUNK = 8704
N_CHUNKS_C = 6
PAD_PAPER = CHUNK * N_CHUNKS_C
PAD_B = CHUNK * 2
WPT = CHUNK // NS
ACC_ROWS = 8720
ZPT = ACC_ROWS // NS
TRASH = CHUNK

BLK = 2048
BATCH = 128
ZVEC = 2048
PER_TILE = E // NS


def _zero_vmem_2d(ref, rows):
    zv = jnp.zeros((LANES,), jnp.float32)

    def body(i, _):
        for h in range(D // LANES):
            ref[i, pl.ds(h * LANES, LANES)] = zv
        return 0

    lax.fori_loop(0, rows, body, 0)


def _zero_vmem_1d(ref, n):
    zv = jnp.zeros((LANES,), jnp.float32)

    def body(i, _):
        ref[pl.ds(i * LANES, LANES)] = zv
        return 0

    lax.fori_loop(0, n // LANES, body, 0)


def _fill_ones_1d(ref, n):
    ov = jnp.ones((LANES,), jnp.float32)
    for i in range(n // LANES):
        ref[pl.ds(i * LANES, LANES)] = ov


def _zero_pass(sid, rows_v2, zvec, acc_sh, cnt_sh):
    zv = jnp.zeros((LANES,), jnp.float32)

    def zrow(i, _):
        for h in range(D // LANES):
            rows_v2[0, i, pl.ds(h * LANES, LANES)] = zv
        return 0

    lax.fori_loop(0, BATCH, zrow, 0)
    _zero_vmem_1d(zvec, ZVEC)
    r0 = sid * ZPT
    done = 0
    while done < ZPT:
        n = min(BATCH, ZPT - done)
        pltpu.sync_copy(rows_v2.at[0, pl.ds(0, n), :],
                        acc_sh.at[pl.ds(r0 + done, n), :])
        done += n
    done = 0
    while done < ZPT * LANES:
        n = min(ZVEC, ZPT * LANES - done)
        pltpu.sync_copy(zvec.at[pl.ds(0, n)],
                        cnt_sh.at[pl.ds(r0 * LANES + done, n)])
        done += n


def _accumulate(sid, x_hbm, e_hbm, lo, hi, trash,
                src_blk, dst_blk, idx_feat, idx_cnt,
                ones_v, rows_v2, acc_sh, cnt_sh, sem):
    e_tile = sid * PER_TILE

    def fire_gather(j):
        pltpu.async_copy(
            x_hbm.at[src_blk.at[pl.ds(j * BATCH, BATCH)]],
            rows_v2.at[lax.rem(j, 2)], sem)

    def wait_gather():
        pltpu.make_async_copy(
            x_hbm.at[pl.ds(0, BATCH), :], rows_v2.at[0], sem).wait()

    def do_block(e0, bs):
        pltpu.sync_copy(e_hbm.at[pl.ds(e0, bs)], src_blk.at[pl.ds(0, bs)])
        pltpu.sync_copy(e_hbm.at[pl.ds(E + e0, bs)], dst_blk.at[pl.ds(0, bs)])
        bs_pad = -(-bs // BATCH) * BATCH
        for g in range((bs_pad - bs) // LANES):
            src_blk[pl.ds(bs + g * LANES, LANES)] = jnp.zeros(
                (LANES,), jnp.int32)
            dst_blk[pl.ds(bs + g * LANES, LANES)] = (
                jnp.zeros((LANES,), jnp.int32) + hi)
        nb = bs_pad // BATCH
        fire_gather(0)

        def bbody(j, _):
            @pl.when(j + 1 < nb)
            def _():
                fire_gather(j + 1)

            jm = lax.rem(j, 2)
            for g in range(BATCH // LANES):
                d_v = dst_blk[pl.ds(j * BATCH + g * LANES, LANES)]
                inb = (d_v >= lo) & (d_v < hi)
                eff = jnp.where(inb, d_v - lo, trash)
                idx_feat[pl.ds(g * LANES, LANES)] = eff
                idx_cnt[pl.ds(g * LANES, LANES)] = eff * LANES
            wait_gather()
            pltpu.sync_copy(rows_v2.at[jm], acc_sh.at[idx_feat], add=True)
            pltpu.sync_copy(ones_v, cnt_sh.at[idx_cnt], add=True)
            return 0

        lax.fori_loop(0, nb, bbody, 0)

    n_full = PER_TILE // BLK

    def blk_body(k, _):
        do_block(e_tile + k * BLK, BLK)
        return 0

    lax.fori_loop(0, n_full, blk_body, 0)
    rem = PER_TILE % BLK
    if rem:
        do_block(e_tile + n_full * BLK, rem)


def _divide_writeout(sid, wpt, out_hbm, out_base,
                     rows_v2, cnt_stage, inv_stage, acc_sh, cnt_sh):
    r0 = sid * wpt
    lane_iota = lax.iota(jnp.int32, LANES)

    done = 0
    while done < wpt:
        n = min(BATCH, wpt - done)
        pltpu.sync_copy(acc_sh.at[pl.ds(r0 + done, n), :],
                        rows_v2.at[0, pl.ds(0, n), :])
        pltpu.sync_copy(cnt_sh.at[pl.ds((r0 + done) * LANES, n * LANES)],
                        cnt_stage.at[pl.ds(0, n * LANES)])

        def inv_body(g, _):
            c16 = plsc.load_gather(
                cnt_stage, [(g * LANES + lane_iota) * LANES])
            inv_stage[pl.ds(g * LANES, LANES)] = 1.0 / jnp.maximum(c16, 1.0)
            return 0

        lax.fori_loop(0, n // LANES, inv_body, 0)

        def row_body(row, _):
            s_vec = plsc.load_gather(
                inv_stage, [jnp.full((LANES,), row, jnp.int32)])
            for h in range(D // LANES):
                rows_v2[0, row, pl.ds(h * LANES, LANES)] = (
                    rows_v2[0, row, pl.ds(h * LANES, LANES)] * s_vec)
            return 0

        lax.fori_loop(0, n, row_body, 0)
        pltpu.sync_copy(rows_v2.at[0, pl.ds(0, n), :],
                        out_hbm.at[pl.ds(out_base + r0 + done, n), :])
        done += n


def _sc_body(x_paper, x_author, e_c, e_w, e_wb,
             mean_c, mean_w, mean_wb,
             src_blk, dst_blk, idx_feat, idx_cnt, ones_v,
             zvec, rows_v2, cnt_stage, inv_stage, acc_sh, cnt_sh, sem):
    cid = lax.axis_index("c")
    sid = lax.axis_index("s")
    _fill_ones_1d(ones_v, BATCH)

    for p in range(3):
        lo = (3 * cid + p) * CHUNK
        hi = lo + CHUNK
        _zero_pass(sid, rows_v2, zvec, acc_sh, cnt_sh)
        plsc.subcore_barrier()
        _accumulate(sid, x_paper, e_c, lo, hi, TRASH,
                    src_blk, dst_blk, idx_feat, idx_cnt,
                    ones_v, rows_v2, acc_sh, cnt_sh, sem)
        plsc.subcore_barrier()
        _divide_writeout(sid, WPT, mean_c, lo,
                         rows_v2, cnt_stage, inv_stage, acc_sh, cnt_sh)
        plsc.subcore_barrier()

    for p in range(2):
        lo = p * CHUNK
        hi = lo + CHUNK
        _zero_pass(sid, rows_v2, zvec, acc_sh, cnt_sh)
        plsc.subcore_barrier()

        @pl.when(cid == 0)
        def _():
            _accumulate(sid, x_author, e_w, lo, hi, TRASH,
                        src_blk, dst_blk, idx_feat, idx_cnt,
                        ones_v, rows_v2, acc_sh, cnt_sh, sem)

        @pl.when(cid == 1)
        def _():
            _accumulate(sid, x_paper, e_wb, lo, hi, TRASH,
                        src_blk, dst_blk, idx_feat, idx_cnt,
                        ones_v, rows_v2, acc_sh, cnt_sh, sem)

        plsc.subcore_barrier()

        @pl.when(cid == 0)
        def _():
            _divide_writeout(sid, WPT, mean_w, lo,
                             rows_v2, cnt_stage, inv_stage, acc_sh, cnt_sh)

        @pl.when(cid == 1)
        def _():
            _divide_writeout(sid, WPT, mean_wb, lo,
                             rows_v2, cnt_stage, inv_stage, acc_sh, cnt_sh)

        plsc.subcore_barrier()


def _segment_means(x_paper, x_author, e_c, e_w, e_wb):
    mesh = plsc.VectorSubcoreMesh(core_axis_name="c", subcore_axis_name="s",
                                  num_cores=NC, num_subcores=NS)
    f32 = jnp.float32
    kfn = pl.kernel(
        _sc_body,
        out_type=(
            jax.ShapeDtypeStruct((PAD_PAPER, D), f32),
            jax.ShapeDtypeStruct((PAD_B, D), f32),
            jax.ShapeDtypeStruct((PAD_B, D), f32),
        ),
        mesh=mesh,
        compiler_params=pltpu.CompilerParams(needs_layout_passes=False),
        scratch_types=[
            pltpu.VMEM((BLK + BATCH,), jnp.int32),
            pltpu.VMEM((BLK + BATCH,), jnp.int32),
            pltpu.VMEM((BATCH,), jnp.int32),
            pltpu.VMEM((BATCH,), jnp.int32),
            pltpu.VMEM((BATCH,), f32),
            pltpu.VMEM((ZVEC,), f32),
            pltpu.VMEM((2, BATCH, D), f32),
            pltpu.VMEM((BATCH * LANES,), f32),
            pltpu.VMEM((BATCH,), f32),
            pltpu.VMEM_SHARED((ACC_ROWS, D), f32),
            pltpu.VMEM_SHARED((ACC_ROWS * LANES,), f32),
            pltpu.SemaphoreType.DMA,
        ],
    )
    return kfn(x_paper, x_author, e_c.reshape(2 * E), e_w.reshape(2 * E),
               e_wb.reshape(2 * E))


BR = 400
N_W_BLOCKS = N_AUTHOR // BR


def _tc_paper_body(x_ref, mc_ref, mw_ref, wr_ref, wnc_ref, wnw_ref, b_ref, o_ref):
    i = pl.program_id(0)
    out = jnp.dot(x_ref[...], wr_ref[...], preferred_element_type=jnp.float32)
    out = out + jnp.dot(mc_ref[...], wnc_ref[...], preferred_element_type=jnp.float32)
    o_ref[...] = out + b_ref[...]

    @pl.when(i < N_W_BLOCKS)
    def _():
        o_ref[...] = o_ref[...] + jnp.dot(
            mw_ref[...], wnw_ref[...], preferred_element_type=jnp.float32)


def _tc_author_body(x_ref, m_ref, wr_ref, wn_ref, b_ref, o_ref):
    out = jnp.dot(x_ref[...], wr_ref[...], preferred_element_type=jnp.float32)
    out = out + jnp.dot(m_ref[...], wn_ref[...], preferred_element_type=jnp.float32)
    o_ref[...] = out + b_ref[...]


def _wspec():
    return pl.BlockSpec((D, D), lambda i: (0, 0))


def _bspec():
    return pl.BlockSpec((1, D), lambda i: (0, 0))


def _tc_paper(x_paper, mean_c, mean_w, wr, wnc, wnw, b):
    grid = (N_PAPER // BR,)
    return pl.pallas_call(
        _tc_paper_body,
        grid=grid,
        in_specs=[
            pl.BlockSpec((BR, D), lambda i: (i, 0)),
            pl.BlockSpec((BR, D), lambda i: (i, 0)),
            pl.BlockSpec((BR, D), lambda i: (jnp.minimum(i, N_W_BLOCKS - 1), 0)),
            _wspec(), _wspec(), _wspec(), _bspec(),
        ],
        out_specs=pl.BlockSpec((BR, D), lambda i: (i, 0)),
        out_shape=jax.ShapeDtypeStruct((N_PAPER, D), jnp.float32),
    )(x_paper, mean_c, mean_w, wr, wnc, wnw, b)


def _tc_author(x_author, mean_wb, wr, wn, b):
    grid = (N_AUTHOR // BR,)
    return pl.pallas_call(
        _tc_author_body,
        grid=grid,
        in_specs=[
            pl.BlockSpec((BR, D), lambda i: (i, 0)),
            pl.BlockSpec((BR, D), lambda i: (i, 0)),
            _wspec(), _wspec(), _bspec(),
        ],
        out_specs=pl.BlockSpec((BR, D), lambda i: (i, 0)),
        out_shape=jax.ShapeDtypeStruct((N_AUTHOR, D), jnp.float32),
    )(x_author, mean_wb, wr, wn, b)


@jax.jit
def kernel(x_paper, x_author, edge_index_cites, edge_index_writes,
           edge_index_written_by,
           W_root_cites, W_neigh_cites, b_cites,
           W_root_writes, W_neigh_writes, b_writes,
           W_root_wb, W_neigh_wb, b_wb):
    mean_c, mean_w, mean_wb = _segment_means(
        x_paper, x_author, edge_index_cites, edge_index_writes,
        edge_index_written_by)
    out_paper = _tc_paper(
        x_paper, mean_c, mean_w,
        W_root_cites + W_root_writes, W_neigh_cites, W_neigh_writes,
        (b_cites + b_writes).reshape(1, D))
    out_author = _tc_author(
        x_author, mean_wb, W_root_wb, W_neigh_wb, b_wb.reshape(1, D))
    return (out_paper, out_author)
